# baseline (speedup 1.0000x reference)
import functools

import jax
import jax.numpy as jnp
from jax import lax
from jax.experimental import pallas as pl
from jax.experimental.pallas import tpu as pltpu
from jax.experimental.pallas import tpu_sc as plsc

NC = 2
NS = 16
NW = NC * NS
CHUNK = 128
BN_EPS = 1e-5


def _einsum_body(x_ref, w_ref, o_ref):
    o_ref[0] = jnp.dot(x_ref[...], w_ref[0], preferred_element_type=jnp.float32)


def _tc_einsum(feat, w):
    n, f = feat.shape
    k, _, fo = w.shape
    bn = 2000
    return pl.pallas_call(
        _einsum_body,
        grid=(n // bn, k),
        in_specs=[
            pl.BlockSpec((bn, f), lambda i, j: (i, 0)),
            pl.BlockSpec((1, f, fo), lambda i, j: (j, 0, 0)),
        ],
        out_specs=pl.BlockSpec((1, bn, fo), lambda i, j: (j, i, 0)),
        out_shape=jax.ShapeDtypeStruct((k, n, fo), jnp.float32),
    )(feat.astype(jnp.bfloat16), w.astype(jnp.bfloat16))


def _bn_silu_body(n, p_ref, g_ref, b_ref, o_ref):
    h = p_ref[0, :n] + p_ref[1, :n]
    mu = jnp.mean(h, axis=0, keepdims=True)
    var = jnp.mean((h - mu) ** 2, axis=0, keepdims=True)
    hn = (h - mu) * lax.rsqrt(var + BN_EPS) * g_ref[0] + b_ref[0]
    o_ref[...] = hn * jax.nn.sigmoid(hn)


def _tc_bn_silu(partials, gamma, beta, n):
    f = partials.shape[2]
    return pl.pallas_call(
        functools.partial(_bn_silu_body, n),
        out_shape=jax.ShapeDtypeStruct((n, f), jnp.float32),
    )(partials, gamma.reshape(1, f), beta.reshape(1, f))


def _final_body(z_ref, wp_ref, bp_ref, gp_ref, bt_ref, p_ref, o_ref):
    zp = jnp.dot(z_ref[...], wp_ref[...], preferred_element_type=jnp.float32)
    zp = zp + bp_ref[0]
    mu = jnp.mean(zp, axis=0, keepdims=True)
    var = jnp.mean((zp - mu) ** 2, axis=0, keepdims=True)
    zp = (zp - mu) * lax.rsqrt(var + BN_EPS) * gp_ref[0] + bt_ref[0]
    zp = jnp.maximum(zp, 0.0)
    n = z_ref.shape[0]
    o_ref[...] = (p_ref[0, :n] + p_ref[1, :n]) + zp


def _tc_final(z, wp, bp, gp, bt, partials):
    n, f = z.shape
    fo = wp.shape[1]
    return pl.pallas_call(
        _final_body,
        out_shape=jax.ShapeDtypeStruct((n, fo), jnp.float32),
    )(z, wp, bp.reshape(1, fo), gp.reshape(1, fo), bt.reshape(1, fo), partials)


def _sc_edge_pass(table, src_slab, ko_slab, dst_slab, zeros, n_nodes, n_acc,
                  nchunk):
    f = table.shape[1]
    mesh = plsc.VectorSubcoreMesh(
        core_axis_name="c", subcore_axis_name="s", num_cores=NC)
    zrows = n_acc // NS

    @functools.partial(
        pl.kernel,
        out_type=jax.ShapeDtypeStruct((NC, n_acc, f), jnp.float32),
        mesh=mesh,
        scratch_types=[
            pltpu.VMEM((nchunk, CHUNK), jnp.int32),
            pltpu.VMEM((nchunk, CHUNK), jnp.int32),
            pltpu.VMEM((nchunk, CHUNK), jnp.int32),
            pltpu.VMEM((CHUNK, f), jnp.float32),
            pltpu.VMEM_SHARED((n_acc, f), jnp.float32),
            pltpu.SemaphoreType.DMA,
        ],
    )
    def body(table_hbm, src_hbm, ko_hbm, dst_hbm, zeros_hbm, out_hbm,
             src_v, ko_v, dst_v, rows_v, acc_sh, sem):
        c = lax.axis_index("c")
        s = lax.axis_index("s")
        wid = c * NS + s

        pltpu.sync_copy(zeros_hbm, acc_sh.at[pl.ds(s * zrows, zrows)])
        pltpu.sync_copy(src_hbm.at[wid], src_v)
        pltpu.sync_copy(ko_hbm.at[wid], ko_v)
        pltpu.sync_copy(dst_hbm.at[wid], dst_v)

        def to_comb(i, carry):
            j = i // 8
            t = (i % 8) * 16
            ko = ko_v[j, pl.ds(t, 16)]
            sv = src_v[j, pl.ds(t, 16)]
            src_v[j, pl.ds(t, 16)] = ko * n_nodes + sv
            return carry
        lax.fori_loop(0, nchunk * 8, to_comb, 0)

        plsc.subcore_barrier()

        def chunk_body(j, carry):
            pltpu.async_copy(table_hbm.at[src_v.at[j]], rows_v, sem).wait()
            pltpu.sync_copy(rows_v, acc_sh.at[dst_v.at[j]], add=True)
            return carry
        lax.fori_loop(0, nchunk, chunk_body, 0)

        plsc.subcore_barrier()

        pltpu.sync_copy(acc_sh.at[pl.ds(s * zrows, zrows)],
                        out_hbm.at[c, pl.ds(s * zrows, zrows)])

    return body(table, src_slab, ko_slab, dst_slab, zeros)


def kernel(x, z, edge_index, kernel_offset, W1, gamma1, beta1, W2, Wp, bp,
           gamma_p, beta_p):
    n, f = x.shape
    e = edge_index.shape[1]
    k = W1.shape[0]

    per_w = -(-e // (NW * CHUNK)) * CHUNK
    nchunk = per_w // CHUNK
    e_pad = per_w * NW
    n_acc = -(-(n + 1) // (NS * 8)) * NS * 8

    pad = e_pad - e
    src = jnp.concatenate([edge_index[0], jnp.zeros((pad,), jnp.int32)])
    ko = jnp.concatenate([kernel_offset, jnp.zeros((pad,), jnp.int32)])
    dst = jnp.concatenate([edge_index[1], jnp.full((pad,), n, jnp.int32)])
    src_slab = src.reshape(NW, nchunk, CHUNK)
    ko_slab = ko.reshape(NW, nchunk, CHUNK)
    dst_slab = dst.reshape(NW, nchunk, CHUNK)
    zeros = jnp.zeros((n_acc // NS, f), jnp.float32)

    y1 = _tc_einsum(x, W1).reshape(k * n, f)
    p1 = _sc_edge_pass(y1, src_slab, ko_slab, dst_slab, zeros, n, n_acc,
                       nchunk)
    h = _tc_bn_silu(p1, gamma1, beta1, n)

    y2 = _tc_einsum(h, W2).reshape(k * n, f)
    p2 = _sc_edge_pass(y2, src_slab, ko_slab, dst_slab, zeros, n, n_acc,
                       nchunk)

    out = _tc_final(z, Wp, bp, gamma_p, beta_p, p2)
    return (out, out)


# speedup vs baseline: 1.1698x; 1.1698x over previous
import functools

import jax
import jax.numpy as jnp
from jax import lax
from jax.experimental import pallas as pl
from jax.experimental.pallas import tpu as pltpu
from jax.experimental.pallas import tpu_sc as plsc

NC = 2
NS = 16
NW = NC * NS
CHUNK = 128
BN_EPS = 1e-5


def _einsum_body(x_ref, w_ref, o_ref):
    o_ref[0] = jnp.dot(x_ref[...], w_ref[0], preferred_element_type=jnp.float32)


def _tc_einsum(feat, w):
    n, f = feat.shape
    k, _, fo = w.shape
    bn = 2000
    return pl.pallas_call(
        _einsum_body,
        grid=(n // bn, k),
        in_specs=[
            pl.BlockSpec((bn, f), lambda i, j: (i, 0)),
            pl.BlockSpec((1, f, fo), lambda i, j: (j, 0, 0)),
        ],
        out_specs=pl.BlockSpec((1, bn, fo), lambda i, j: (j, i, 0)),
        out_shape=jax.ShapeDtypeStruct((k, n, fo), jnp.float32),
    )(feat, w)


def _bn_silu_body(n, p_ref, g_ref, b_ref, o_ref):
    h = p_ref[0, :n] + p_ref[1, :n]
    mu = jnp.mean(h, axis=0, keepdims=True)
    var = jnp.mean((h - mu) ** 2, axis=0, keepdims=True)
    hn = (h - mu) * lax.rsqrt(var + BN_EPS) * g_ref[0] + b_ref[0]
    o_ref[...] = hn * jax.nn.sigmoid(hn)


def _tc_bn_silu(partials, gamma, beta, n):
    f = partials.shape[2]
    return pl.pallas_call(
        functools.partial(_bn_silu_body, n),
        out_shape=jax.ShapeDtypeStruct((n, f), jnp.float32),
    )(partials, gamma.reshape(1, f), beta.reshape(1, f))


def _final_body(z_ref, wp_ref, bp_ref, gp_ref, bt_ref, p_ref, o_ref):
    zp = jnp.dot(z_ref[...], wp_ref[...], preferred_element_type=jnp.float32)
    zp = zp + bp_ref[0]
    mu = jnp.mean(zp, axis=0, keepdims=True)
    var = jnp.mean((zp - mu) ** 2, axis=0, keepdims=True)
    zp = (zp - mu) * lax.rsqrt(var + BN_EPS) * gp_ref[0] + bt_ref[0]
    zp = jnp.maximum(zp, 0.0)
    n = z_ref.shape[0]
    o_ref[...] = (p_ref[0, :n] + p_ref[1, :n]) + zp


def _tc_final(z, wp, bp, gp, bt, partials):
    n, f = z.shape
    fo = wp.shape[1]
    return pl.pallas_call(
        _final_body,
        out_shape=jax.ShapeDtypeStruct((n, fo), jnp.float32),
    )(z, wp, bp.reshape(1, fo), gp.reshape(1, fo), bt.reshape(1, fo), partials)


def _sc_edge_pass(table, src_slab, ko_slab, dst_slab, zeros, n_nodes, n_acc,
                  nchunk):
    f = table.shape[1]
    mesh = plsc.VectorSubcoreMesh(
        core_axis_name="c", subcore_axis_name="s", num_cores=NC)
    zrows = n_acc // NS

    @functools.partial(
        pl.kernel,
        out_type=jax.ShapeDtypeStruct((NC, n_acc, f), jnp.float32),
        mesh=mesh,
        scratch_types=[
            pltpu.VMEM((nchunk, CHUNK), jnp.int32),
            pltpu.VMEM((nchunk, CHUNK), jnp.int32),
            pltpu.VMEM((nchunk, CHUNK), jnp.int32),
            pltpu.VMEM((CHUNK, f), jnp.float32),
            pltpu.VMEM_SHARED((n_acc, f), jnp.float32),
            pltpu.SemaphoreType.DMA,
        ],
    )
    def body(table_hbm, src_hbm, ko_hbm, dst_hbm, zeros_hbm, out_hbm,
             src_v, ko_v, dst_v, rows_v, acc_sh, sem):
        c = lax.axis_index("c")
        s = lax.axis_index("s")
        wid = c * NS + s

        pltpu.sync_copy(zeros_hbm, acc_sh.at[pl.ds(s * zrows, zrows)])
        pltpu.sync_copy(src_hbm.at[wid], src_v)
        pltpu.sync_copy(ko_hbm.at[wid], ko_v)
        pltpu.sync_copy(dst_hbm.at[wid], dst_v)

        def to_comb(i, carry):
            j = i // 8
            t = (i % 8) * 16
            ko = ko_v[j, pl.ds(t, 16)]
            sv = src_v[j, pl.ds(t, 16)]
            src_v[j, pl.ds(t, 16)] = ko * n_nodes + sv
            return carry
        lax.fori_loop(0, nchunk * 8, to_comb, 0)

        plsc.subcore_barrier()

        def chunk_body(j, carry):
            pltpu.async_copy(table_hbm.at[src_v.at[j]], rows_v, sem).wait()
            pltpu.sync_copy(rows_v, acc_sh.at[dst_v.at[j]], add=True)
            return carry
        lax.fori_loop(0, nchunk, chunk_body, 0)

        plsc.subcore_barrier()

        pltpu.sync_copy(acc_sh.at[pl.ds(s * zrows, zrows)],
                        out_hbm.at[c, pl.ds(s * zrows, zrows)])

    return body(table, src_slab, ko_slab, dst_slab, zeros)


def kernel(x, z, edge_index, kernel_offset, W1, gamma1, beta1, W2, Wp, bp,
           gamma_p, beta_p):
    n, f = x.shape
    e = edge_index.shape[1]
    k = W1.shape[0]

    per_w = -(-e // (NW * CHUNK)) * CHUNK
    nchunk = per_w // CHUNK
    e_pad = per_w * NW
    n_acc = -(-(n + 1) // (NS * 8)) * NS * 8

    pad = e_pad - e
    src = jnp.concatenate([edge_index[0], jnp.zeros((pad,), jnp.int32)])
    ko = jnp.concatenate([kernel_offset, jnp.zeros((pad,), jnp.int32)])
    dst = jnp.concatenate([edge_index[1], jnp.full((pad,), n, jnp.int32)])
    src_slab = src.reshape(NW, nchunk, CHUNK)
    ko_slab = ko.reshape(NW, nchunk, CHUNK)
    dst_slab = dst.reshape(NW, nchunk, CHUNK)
    zeros = jnp.zeros((n_acc // NS, f), jnp.float32)

    y1 = _tc_einsum(x, W1).reshape(k * n, f)
    p1 = _sc_edge_pass(y1, src_slab, ko_slab, dst_slab, zeros, n, n_acc,
                       nchunk)
    h = _tc_bn_silu(p1, gamma1, beta1, n)

    y2 = _tc_einsum(h, W2).reshape(k * n, f)
    p2 = _sc_edge_pass(y2, src_slab, ko_slab, dst_slab, zeros, n, n_acc,
                       nchunk)

    out = _tc_final(z, Wp, bp, gamma_p, beta_p, p2)
    return (out, out)
